# Initial kernel scaffold; baseline (speedup 1.0000x reference)
import jax
import jax.numpy as jnp
from jax.experimental import pallas as pl


def kernel(x, edge_index, gamma, beta, run_mean, run_var, W_in, b_in, W_l, b_l, att, conv_b, gn_w, gn_b, gn_ms, get_attention_weights):
    raise NotImplementedError("write your pallas kernel here")


# speedup vs baseline: 4.2791x; 4.2791x over previous
import functools

import jax
import jax.numpy as jnp
from jax import lax
from jax.experimental import pallas as pl
from jax.experimental.pallas import tpu as pltpu
from jax.experimental.pallas import tpu_sc as plsc

N_NODES = 10000
D_FEAT = 128
D_EMB = 64
N_LAYERS = 4
N_EDGES_IN = 320000
E_TOT = N_EDGES_IN + N_NODES
NPAD = 10240
E_PAD = 331776
NW = 32
CHUNK = E_PAD // NW
BLK = 384
NBLK = CHUNK // BLK
GPB = BLK // 16
SL = NPAD // 16

_mesh = plsc.VectorSubcoreMesh(core_axis_name="c", subcore_axis_name="s")


def _leaky(v, s):
    return jnp.maximum(v, 0.0) + s * jnp.minimum(v, 0.0)


def _tc_front_body(x_ref, gm_ref, bt_ref, mu_ref, vr_ref, wi_ref, bi_ref,
                   w0_ref, b0_ref, xl64_ref, xl80_ref):
    x = x_ref[...]
    inv = lax.rsqrt(vr_ref[...] + 1e-5)
    h = (x - mu_ref[...]) * inv * gm_ref[...] + bt_ref[...]
    h = _leaky(jnp.dot(h, wi_ref[...], preferred_element_type=jnp.float32)
               + bi_ref[...], 0.01)
    xl = jnp.dot(h, w0_ref[...], preferred_element_type=jnp.float32) + b0_ref[...]
    xl64_ref[...] = jnp.zeros((NPAD, D_EMB), jnp.float32)
    xl64_ref[0:N_NODES, :] = xl
    xl80_ref[...] = jnp.zeros((NPAD, 80), jnp.float32)
    xl80_ref[0:N_NODES, 0:D_EMB] = xl
    xl80_ref[0:N_NODES, D_EMB:D_EMB + 1] = jnp.ones((N_NODES, 1), jnp.float32)


def _tc_front(x, gamma, beta, run_mean, run_var, W_in, b_in, W0, b0):
    return pl.pallas_call(
        _tc_front_body,
        out_shape=(jax.ShapeDtypeStruct((NPAD, D_EMB), jnp.float32),
                   jax.ShapeDtypeStruct((NPAD, 80), jnp.float32)),
    )(x, gamma.reshape(1, -1), beta.reshape(1, -1), run_mean.reshape(1, -1),
      run_var.reshape(1, -1), W_in, b_in.reshape(1, -1), W0, b0.reshape(1, -1))


def _graphnorm(aggu_ref, cb_ref, gw_ref, gb_ref, gms_ref):
    a = aggu_ref[0] + aggu_ref[1]
    den = a[0:N_NODES, D_EMB:D_EMB + 1]
    agg = a[0:N_NODES, 0:D_EMB] / (den + 1e-16) + cb_ref[...]
    mean = jnp.mean(agg, axis=0, keepdims=True)
    out = agg - mean * gms_ref[...]
    var = jnp.mean(out * out, axis=0, keepdims=True)
    h = gw_ref[...] * out * lax.rsqrt(var + 1e-5) + gb_ref[...]
    return _leaky(h, 0.01), a[:, D_EMB:D_EMB + 1]


def _tc_mid_body(aggu_ref, cb_ref, gw_ref, gb_ref, gms_ref, w_ref, b_ref,
                 xl64_ref, xl80_ref, den_ref):
    h, den_all = _graphnorm(aggu_ref, cb_ref, gw_ref, gb_ref, gms_ref)
    xl = jnp.dot(h, w_ref[...], preferred_element_type=jnp.float32) + b_ref[...]
    xl64_ref[...] = jnp.zeros((NPAD, D_EMB), jnp.float32)
    xl64_ref[0:N_NODES, :] = xl
    xl80_ref[...] = jnp.zeros((NPAD, 80), jnp.float32)
    xl80_ref[0:N_NODES, 0:D_EMB] = xl
    xl80_ref[0:N_NODES, D_EMB:D_EMB + 1] = jnp.ones((N_NODES, 1), jnp.float32)
    den_ref[...] = den_all


def _tc_mid(aggu, conv_b, gn_w, gn_b, gn_ms, W, b):
    return pl.pallas_call(
        _tc_mid_body,
        out_shape=(jax.ShapeDtypeStruct((NPAD, D_EMB), jnp.float32),
                   jax.ShapeDtypeStruct((NPAD, 80), jnp.float32),
                   jax.ShapeDtypeStruct((NPAD, 1), jnp.float32)),
    )(aggu, conv_b.reshape(1, -1), gn_w.reshape(1, -1), gn_b.reshape(1, -1),
      gn_ms.reshape(1, -1), W, b.reshape(1, -1))


def _tc_last_body(aggu_ref, cb_ref, gw_ref, gb_ref, gms_ref, h_ref, den_ref):
    h, den_all = _graphnorm(aggu_ref, cb_ref, gw_ref, gb_ref, gms_ref)
    h_ref[...] = h
    den_ref[...] = den_all


def _tc_last(aggu, conv_b, gn_w, gn_b, gn_ms):
    return pl.pallas_call(
        _tc_last_body,
        out_shape=(jax.ShapeDtypeStruct((N_NODES, D_EMB), jnp.float32),
                   jax.ShapeDtypeStruct((NPAD, 1), jnp.float32)),
    )(aggu, conv_b.reshape(1, -1), gn_w.reshape(1, -1), gn_b.reshape(1, -1),
      gn_ms.reshape(1, -1))


def _k1_body(xl64_hbm, src_hbm, dst_hbm, att_hbm, e_out, m_out,
             idx_a, idx_b, rows_a, rows_b, e_blk, m_loc, att_v, m_ca, m_cb,
             m_sh, sem):
    c = lax.axis_index("c")
    s = lax.axis_index("s")
    wid = s * 2 + c
    base = pl.multiple_of(wid * CHUNK, 8)
    pltpu.sync_copy(att_hbm, att_v)

    neg = jnp.full((16,), -3e38, jnp.float32)

    def init_b(i, _):
        m_loc[pl.ds(i * 16, 16)] = neg
        return 0
    lax.fori_loop(0, NPAD // 16, init_b, 0)

    att_c = [att_v[pl.ds(k * 16, 16)] for k in range(D_EMB // 16)]

    def blk_body(b, _):
        off = pl.multiple_of(base + b * BLK, 8)
        pltpu.sync_copy(src_hbm.at[pl.ds(off, BLK)], idx_a)
        pltpu.sync_copy(dst_hbm.at[pl.ds(off, BLK)], idx_b)
        pltpu.async_copy(xl64_hbm.at[idx_a], rows_a, sem).wait()
        pltpu.async_copy(xl64_hbm.at[idx_b], rows_b, sem).wait()

        def grp(g, _):
            ids = g * 16 + lax.iota(jnp.int32, 16)
            acc = jnp.zeros((16,), jnp.float32)
            for d in range(D_EMB):
                dd = jnp.full((16,), d, jnp.int32)
                v = (plsc.load_gather(rows_a, [ids, dd])
                     + plsc.load_gather(rows_b, [ids, dd]))
                acc = acc + att_c[d // 16][d % 16] * _leaky(v, 0.2)
            e_blk[pl.ds(g * 16, 16)] = acc

            dst_v = idx_b[pl.ds(g * 16, 16)]

            def upd_cond(pending):
                return jnp.any(pending)

            def upd_body(pending):
                cur = plsc.load_gather(m_loc, [dst_v])
                plsc.store_scatter(m_loc, [dst_v],
                                   jnp.maximum(cur, acc), mask=pending)
                chk = plsc.load_gather(m_loc, [dst_v])
                return pending & (chk < acc)
            lax.while_loop(upd_cond, upd_body,
                           jnp.ones((16,), jnp.bool_))
            return 0
        lax.fori_loop(0, GPB, grp, 0)
        pltpu.sync_copy(e_blk, e_out.at[pl.ds(off, BLK)])
        return 0
    lax.fori_loop(0, NBLK, blk_body, 0)

    pltpu.sync_copy(m_loc, m_sh.at[s])
    plsc.subcore_barrier()
    colo = pl.multiple_of(s * SL, 8)
    pltpu.sync_copy(m_sh.at[0, pl.ds(colo, SL)], m_ca)

    def red_j(j, _):
        pltpu.sync_copy(m_sh.at[j, pl.ds(colo, SL)], m_cb)

        def mx(k, _):
            sl = pl.ds(k * 16, 16)
            m_ca[sl] = jnp.maximum(m_ca[sl], m_cb[sl])
            return 0
        lax.fori_loop(0, SL // 16, mx, 0)
        return 0
    lax.fori_loop(1, 16, red_j, 0)
    pltpu.sync_copy(m_ca, m_out.at[c, pl.ds(colo, SL)])


@functools.partial(
    pl.kernel, mesh=_mesh,
    compiler_params=pltpu.CompilerParams(needs_layout_passes=False, use_tc_tiling_on_sc=False),
    out_type=(jax.ShapeDtypeStruct((E_PAD,), jnp.float32),
              jax.ShapeDtypeStruct((2, NPAD), jnp.float32)),
    scratch_types=[
        pltpu.VMEM((BLK,), jnp.int32),
        pltpu.VMEM((BLK,), jnp.int32),
        pltpu.VMEM((BLK, D_EMB), jnp.float32),
        pltpu.VMEM((BLK, D_EMB), jnp.float32),
        pltpu.VMEM((BLK,), jnp.float32),
        pltpu.VMEM((NPAD,), jnp.float32),
        pltpu.VMEM((D_EMB,), jnp.float32),
        pltpu.VMEM((SL,), jnp.float32),
        pltpu.VMEM((SL,), jnp.float32),
        pltpu.VMEM_SHARED((16, NPAD), jnp.float32),
        pltpu.SemaphoreType.DMA,
    ])
def _k1(xl64_hbm, src_hbm, dst_hbm, att_hbm, e_out, m_out, *scratch):
    _k1_body(xl64_hbm, src_hbm, dst_hbm, att_hbm, e_out, m_out, *scratch)


def _k2_body(xl80_hbm, src_hbm, dst_hbm, e_hbm, m_hbm, ee_out, agg_out,
             idx_a, idx_b, rows, e_blk, m_loc, m_tmp, agg_sh, sem):
    c = lax.axis_index("c")
    s = lax.axis_index("s")
    wid = s * 2 + c
    base = pl.multiple_of(wid * CHUNK, 8)

    pltpu.sync_copy(m_hbm.at[0], m_loc)
    pltpu.sync_copy(m_hbm.at[1], m_tmp)

    def mx(k, _):
        sl = pl.ds(k * 16, 16)
        m_loc[sl] = jnp.maximum(m_loc[sl], m_tmp[sl])
        return 0
    lax.fori_loop(0, NPAD // 16, mx, 0)

    zero16 = jnp.zeros((16,), jnp.float32)

    def zr(r, _):
        def zc(k, _):
            rows[r, pl.ds(k * 16, 16)] = zero16
            return 0
        lax.fori_loop(0, 5, zc, 0)
        return 0
    lax.fori_loop(0, BLK, zr, 0)
    rowo = pl.multiple_of(s * SL, 8)
    pltpu.sync_copy(rows, agg_sh.at[pl.ds(rowo, BLK), :])
    pltpu.sync_copy(rows.at[pl.ds(0, SL - BLK), :],
                    agg_sh.at[pl.ds(rowo + BLK, SL - BLK), :])
    plsc.subcore_barrier()

    def blk_body(b, _):
        off = pl.multiple_of(base + b * BLK, 8)
        pltpu.sync_copy(src_hbm.at[pl.ds(off, BLK)], idx_a)
        pltpu.sync_copy(dst_hbm.at[pl.ds(off, BLK)], idx_b)
        pltpu.sync_copy(e_hbm.at[pl.ds(off, BLK)], e_blk)
        pltpu.async_copy(xl80_hbm.at[idx_a], rows, sem).wait()

        def grp(g, _):
            sl = pl.ds(g * 16, 16)
            ids = g * 16 + lax.iota(jnp.int32, 16)
            dst_v = idx_b[sl]
            mg = plsc.load_gather(m_loc, [dst_v])
            ee = jnp.exp(e_blk[sl] - mg)
            e_blk[sl] = ee

            def dscale(d, _):
                dd = jnp.full((16,), d, jnp.int32)
                col = plsc.load_gather(rows, [ids, dd])
                plsc.store_scatter(rows, [ids, dd], col * ee)
                return 0
            lax.fori_loop(0, D_EMB + 1, dscale, 0)
            return 0
        lax.fori_loop(0, GPB, grp, 0)
        pltpu.sync_copy(rows, agg_sh.at[idx_b], add=True)
        pltpu.sync_copy(e_blk, ee_out.at[pl.ds(off, BLK)])
        return 0
    lax.fori_loop(0, NBLK, blk_body, 0)

    plsc.subcore_barrier()
    pltpu.sync_copy(agg_sh.at[pl.ds(rowo, SL), :],
                    agg_out.at[c, pl.ds(rowo, SL), :])


@functools.partial(
    pl.kernel, mesh=_mesh,
    compiler_params=pltpu.CompilerParams(needs_layout_passes=False, use_tc_tiling_on_sc=False),
    out_type=(jax.ShapeDtypeStruct((E_PAD,), jnp.float32),
              jax.ShapeDtypeStruct((2, NPAD, 80), jnp.float32)),
    scratch_types=[
        pltpu.VMEM((BLK,), jnp.int32),
        pltpu.VMEM((BLK,), jnp.int32),
        pltpu.VMEM((BLK, 80), jnp.float32),
        pltpu.VMEM((BLK,), jnp.float32),
        pltpu.VMEM((NPAD,), jnp.float32),
        pltpu.VMEM((NPAD,), jnp.float32),
        pltpu.VMEM_SHARED((NPAD, 80), jnp.float32),
        pltpu.SemaphoreType.DMA,
    ])
def _k2(xl80_hbm, src_hbm, dst_hbm, e_hbm, m_hbm, ee_out, agg_out, *scratch):
    _k2_body(xl80_hbm, src_hbm, dst_hbm, e_hbm, m_hbm, ee_out, agg_out,
             *scratch)


def _k3_body(ee_hbm, dst_hbm, den_hbm, alpha_out, idx_b, ee_blk, den_loc):
    c = lax.axis_index("c")
    s = lax.axis_index("s")
    wid = s * 2 + c
    base = pl.multiple_of(wid * CHUNK, 8)
    for l in range(N_LAYERS):
        pltpu.sync_copy(den_hbm.at[l], den_loc)

        def blk_body(b, _):
            off = pl.multiple_of(base + b * BLK, 8)
            pltpu.sync_copy(dst_hbm.at[pl.ds(off, BLK)], idx_b)
            pltpu.sync_copy(ee_hbm.at[l, pl.ds(off, BLK)], ee_blk)

            def grp(g, _):
                sl = pl.ds(g * 16, 16)
                dn = plsc.load_gather(den_loc, [idx_b[sl]])
                ee_blk[sl] = ee_blk[sl] / (dn + 1e-16)
                return 0
            lax.fori_loop(0, GPB, grp, 0)
            pltpu.sync_copy(ee_blk, alpha_out.at[l, pl.ds(off, BLK)])
            return 0
        lax.fori_loop(0, NBLK, blk_body, 0)


@functools.partial(
    pl.kernel, mesh=_mesh,
    compiler_params=pltpu.CompilerParams(needs_layout_passes=False, use_tc_tiling_on_sc=False),
    out_type=jax.ShapeDtypeStruct((N_LAYERS, E_PAD), jnp.float32),
    scratch_types=[
        pltpu.VMEM((BLK,), jnp.int32),
        pltpu.VMEM((BLK,), jnp.float32),
        pltpu.VMEM((NPAD,), jnp.float32),
    ])
def _k3(ee_hbm, dst_hbm, den_hbm, alpha_out, *scratch):
    _k3_body(ee_hbm, dst_hbm, den_hbm, alpha_out, *scratch)


def kernel(x, edge_index, gamma, beta, run_mean, run_var, W_in, b_in,
           W_l, b_l, att, conv_b, gn_w, gn_b, gn_ms, get_attention_weights):
    loops = jnp.arange(N_NODES, dtype=jnp.int32)
    padv = jnp.full((E_PAD - E_TOT,), N_NODES, jnp.int32)
    src = jnp.concatenate([edge_index[0].astype(jnp.int32), loops, padv])
    dst = jnp.concatenate([edge_index[1].astype(jnp.int32), loops, padv])

    xl64, xl80 = _tc_front(x, gamma, beta, run_mean, run_var, W_in, b_in,
                           W_l[0], b_l[0])
    ee_list, den_list = [], []
    h = None
    for l in range(N_LAYERS):
        e, m = _k1(xl64, src, dst, att[l])
        ee, aggu = _k2(xl80, src, dst, e, m)
        ee_list.append(ee)
        if l + 1 < N_LAYERS:
            xl64, xl80, den = _tc_mid(aggu, conv_b[l], gn_w[l], gn_b[l],
                                      gn_ms[l], W_l[l + 1], b_l[l + 1])
        else:
            h, den = _tc_last(aggu, conv_b[l], gn_w[l], gn_b[l], gn_ms[l])
        den_list.append(den)

    ee_all = jnp.stack(ee_list)
    den_all = jnp.stack([d.reshape(NPAD) for d in den_list])
    alpha = _k3(ee_all, dst, den_all)
    attns = alpha[:, :E_TOT]
    return (h, h, attns)
